# baseline (speedup 1.0000x reference)
import jax
import jax.numpy as jnp
from jax import lax
from jax.experimental import pallas as pl
from jax.experimental.pallas import tpu as pltpu
from jax.experimental.pallas import tpu_sc as plsc

_NC = 2
_NS = 16
_NW = _NC * _NS

_CHUNK = 100
_NBUF = 8
_LAG = 6


def _make_sc_gather(bsz: int, seq: int, dim: int):
    assert seq % (2 * _CHUNK) == 0 and bsz % _NW == 0
    b_per_w = bsz // _NW
    steps_per_b = seq // _CHUNK
    nsteps = b_per_w * steps_per_b
    assert nsteps % _NBUF == 0 and _LAG < _NBUF

    mesh = plsc.VectorSubcoreMesh(core_axis_name="c", subcore_axis_name="s")

    scratch = [
        pltpu.VMEM((nsteps, _CHUNK), jnp.int32),
        pltpu.VMEM((_NBUF, _CHUNK, dim), jnp.float32),
    ] + [pltpu.SemaphoreType.DMA] * (2 * _NBUF)

    def body(idx_hbm, table_hbm, out_hbm, idx_v, rows_v, *sems):
        gsem = sems[:_NBUF]
        ssem = sems[_NBUF:]
        wid = lax.axis_index("s") * _NC + lax.axis_index("c")
        b_base = wid * b_per_w

        pltpu.sync_copy(idx_hbm.at[pl.ds(wid * nsteps, nsteps)], idx_v)

        def out_slice(j):
            b = b_base + j // steps_per_b
            t0 = (j % steps_per_b) * _CHUNK
            return out_hbm.at[b, pl.ds(t0, _CHUNK)]

        def gather_start(j, b):
            pltpu.async_copy(table_hbm.at[idx_v.at[j]], rows_v.at[b], gsem[b])

        def gather_wait(j, b):
            pltpu.make_async_copy(
                table_hbm.at[idx_v.at[j]], rows_v.at[b], gsem[b]).wait()

        def scatter_start(j, b):
            pltpu.async_copy(rows_v.at[b], out_slice(j), ssem[b])

        def scatter_wait(j, b):
            pltpu.make_async_copy(rows_v.at[b], out_slice(j), ssem[b]).wait()

        for j in range(_NBUF):
            gather_start(j, j)
            if j >= _LAG:
                jc = j - _LAG
                gather_wait(jc, jc)
                scatter_start(jc, jc)

        def outer_body(g, _):
            for b in range(_NBUF):
                j = g * _NBUF + b
                scatter_wait(j - _NBUF, b)
                gather_start(j, b)
                jc = j - _LAG
                bc = (b - _LAG) % _NBUF
                gather_wait(jc, bc)
                scatter_start(jc, bc)
            return 0

        lax.fori_loop(1, nsteps // _NBUF, outer_body, 0)

        for jc in range(nsteps - _LAG, nsteps):
            gather_wait(jc, jc % _NBUF)
            scatter_start(jc, jc % _NBUF)
        for j in range(nsteps - _NBUF, nsteps):
            scatter_wait(j, j % _NBUF)

    return pl.kernel(
        body,
        out_type=jax.ShapeDtypeStruct((bsz, seq, dim), jnp.float32),
        mesh=mesh,
        scratch_types=scratch,
        compiler_params=pltpu.CompilerParams(use_tc_tiling_on_sc=False),
    )


@jax.jit
def kernel(token_ids, weight):
    bsz, seq = token_ids.shape
    num, dim = weight.shape
    idx = token_ids.astype(jnp.int32)
    idx2d = idx.reshape(bsz * seq // _CHUNK, _CHUNK)
    return _make_sc_gather(bsz, seq, dim)(idx2d, weight)


# speedup vs baseline: 1.3328x; 1.3328x over previous
import jax
import jax.numpy as jnp
from jax import lax
from jax.experimental import pallas as pl
from jax.experimental.pallas import tpu as pltpu
from jax.experimental.pallas import tpu_sc as plsc

_NC = 2
_NS = 16
_NW = _NC * _NS

_NBUF = 4
_LAG = 2
_UNROLL = 16


def _make_sc_gather(bsz: int, seq: int, dim: int):
    assert bsz % _NW == 0 and seq >= _UNROLL
    b_per_w = bsz // _NW
    assert b_per_w % _NBUF == 0 and _LAG < _NBUF

    mesh = plsc.VectorSubcoreMesh(core_axis_name="c", subcore_axis_name="s")

    scratch = [
        pltpu.VMEM((_NBUF, seq, dim), jnp.float32),
        pltpu.VMEM((_NBUF, seq), jnp.int32),
    ] + [pltpu.SemaphoreType.DMA] * (2 * _NBUF)

    def body(idx_hbm, table_hbm, out_hbm, rows_v, idx_v, *sems):
        gsem = sems[:_NBUF]
        ssem = sems[_NBUF:]
        wid = lax.axis_index("s") * _NC + lax.axis_index("c")
        b_base = wid * b_per_w

        def gather_start(c, s):
            pltpu.sync_copy(idx_hbm.at[b_base + c], idx_v.at[s])

            ngroups = seq // _UNROLL
            tail = seq - ngroups * _UNROLL

            def issue(g, _):
                toks = idx_v[s, pl.ds(g * _UNROLL, _UNROLL)]
                for u in range(_UNROLL):
                    pltpu.async_copy(
                        table_hbm.at[toks[u]],
                        rows_v.at[s, g * _UNROLL + u], gsem[s])
                return 0

            lax.fori_loop(0, ngroups, issue, 0)
            if tail:
                toks = idx_v[s, pl.ds(seq - _UNROLL, _UNROLL)]
                for u in range(_UNROLL - tail, _UNROLL):
                    pltpu.async_copy(
                        table_hbm.at[toks[u]],
                        rows_v.at[s, seq - _UNROLL + u], gsem[s])

        def gather_wait(s):
            pltpu.make_async_copy(
                table_hbm.at[pl.ds(0, seq)], rows_v.at[s], gsem[s]).wait()

        def scatter_start(c, s):
            pltpu.async_copy(rows_v.at[s], out_hbm.at[b_base + c], ssem[s])

        def scatter_wait(c, s):
            pltpu.make_async_copy(
                rows_v.at[s], out_hbm.at[b_base + c], ssem[s]).wait()

        for c in range(_NBUF):
            gather_start(c, c)
            if c >= _LAG:
                cc = c - _LAG
                gather_wait(cc)
                scatter_start(cc, cc)

        def outer_body(g, _):
            for s in range(_NBUF):
                c = g * _NBUF + s
                scatter_wait(c - _NBUF, s)
                gather_start(c, s)
                cc = c - _LAG
                sc = (s - _LAG) % _NBUF
                gather_wait(sc)
                scatter_start(cc, sc)
            return 0

        lax.fori_loop(1, b_per_w // _NBUF, outer_body, 0)

        for cc in range(b_per_w - _LAG, b_per_w):
            gather_wait(cc % _NBUF)
            scatter_start(cc, cc % _NBUF)
        for c in range(b_per_w - _NBUF, b_per_w):
            scatter_wait(c, c % _NBUF)

    return pl.kernel(
        body,
        out_type=jax.ShapeDtypeStruct((bsz, seq, dim), jnp.float32),
        mesh=mesh,
        scratch_types=scratch,
        compiler_params=pltpu.CompilerParams(use_tc_tiling_on_sc=True),
    )


@jax.jit
def kernel(token_ids, weight):
    bsz, seq = token_ids.shape
    num, dim = weight.shape
    idx = token_ids.astype(jnp.int32)
    return _make_sc_gather(bsz, seq, dim)(idx, weight)
